# Initial kernel scaffold; baseline (speedup 1.0000x reference)
import jax
import jax.numpy as jnp
from jax.experimental import pallas as pl


def kernel(x, edge_index, edge_attr, pos, batch, W1, root1, bias1, g1, be1, W2, root2, bias2, g2, be2, W3, root3, bias3, g3, be3, W4, root4, bias4, g4, be4, fc1_w, fc1_b, fc2_w, fc2_b):
    raise NotImplementedError("write your pallas kernel here")


# speedup vs baseline: 1.0001x; 1.0001x over previous
import functools

import jax
import jax.numpy as jnp
from jax.experimental import pallas as pl

K = 5
B = 8


def _spline_basis(pseudo):
    v = jnp.clip(pseudo, 0.0, 1.0) * (K - 1)
    f = jnp.clip(jnp.floor(v), 0, K - 2)
    frac = v - f
    fi = f.astype(jnp.int32)
    bas, wis = [], []
    for b in range(8):
        w = jnp.ones((pseudo.shape[0],), pseudo.dtype)
        idx = jnp.zeros((pseudo.shape[0],), jnp.int32)
        for d in range(3):
            bit = (b >> d) & 1
            w = w * (frac[:, d] if bit else (1.0 - frac[:, d]))
            idx = idx + (fi[:, d] + bit) * (K ** d)
        bas.append(w)
        wis.append(idx)
    return jnp.stack(bas, 1), jnp.stack(wis, 1)


def _spline_conv(x, src, dst, pseudo, emask, W, root, bias):
    N, Cin = x.shape
    Cout = W.shape[2]
    basis, wi = _spline_basis(pseudo)
    basis = basis * emask[:, None]
    xs = x[src]
    T = jnp.zeros((N * 125, Cin), x.dtype)
    for b in range(8):
        T = T.at[dst * 125 + wi[:, b]].add(basis[:, b][:, None] * xs)
    out = T.reshape(N, 125 * Cin) @ W.reshape(125 * Cin, Cout)
    deg = jnp.zeros((N,), x.dtype).at[dst].add(emask)
    out = out / jnp.clip(deg, 1.0, None)[:, None]
    return out + x @ root + bias


def _bn(x, g, b):
    m = jnp.mean(x, 0)
    v = jnp.var(x, 0)
    return g * (x - m) / jnp.sqrt(v + 1e-5) + b


def _voxel(pos, batch, size, nvox):
    c = jnp.clip(jnp.floor(pos / size).astype(jnp.int32), 0, nvox - 1)
    local = c[:, 0] + c[:, 1] * nvox + c[:, 2] * nvox * nvox
    return batch * (nvox ** 3) + local


def _max_pool(cluster, M, x, pos, src, dst, emask, nvox):
    xn = jax.ops.segment_max(x, cluster, num_segments=M)
    xn = jnp.where(jnp.isfinite(xn), xn, 0.0)
    cnt = jnp.zeros((M,), x.dtype).at[cluster].add(1.0)
    pn = jnp.zeros((M, 3), pos.dtype).at[cluster].add(pos) / jnp.clip(cnt, 1.0, None)[:, None]
    bnew = (jnp.arange(M, dtype=jnp.int32) // (nvox ** 3))
    sn = cluster[src]
    dn = cluster[dst]
    em = emask * (sn != dn).astype(x.dtype)
    delta = pn[dn] - pn[sn]
    mx = jnp.max(jnp.abs(delta))
    pseudo = delta / (2.0 * mx + 1e-12) + 0.5
    return xn, pn, bnew, sn, dn, em, pseudo


def _fc1_body(z_ref, w_ref, acc_ref):
    @pl.when(pl.program_id(0) == 0)
    def _():
        acc_ref[...] = jnp.zeros_like(acc_ref)
    acc_ref[...] += jnp.dot(z_ref[...], w_ref[...],
                            preferred_element_type=jnp.float32)


def _head_body(acc_ref, b1_ref, w2_ref, b2_ref, out_ref):
    h = acc_ref[...] + b1_ref[...]
    h = jnp.where(h > 0, h, jnp.exp(jnp.minimum(h, 0.0)) - 1.0)
    y = jnp.dot(h, w2_ref[...], preferred_element_type=jnp.float32) + b2_ref[...]
    mx = jnp.max(y, axis=1, keepdims=True)
    lse = jnp.log(jnp.sum(jnp.exp(y - mx), axis=1, keepdims=True)) + mx
    out_ref[...] = y - lse


def _fc_head(z, fc1_w, fc1_b, fc2_w, fc2_b):
    KC = 2048
    nk = z.shape[1] // KC
    acc = pl.pallas_call(
        _fc1_body,
        grid=(nk,),
        in_specs=[
            pl.BlockSpec((B, KC), lambda k: (0, k)),
            pl.BlockSpec((KC, 1024), lambda k: (k, 0)),
        ],
        out_specs=pl.BlockSpec((B, 1024), lambda k: (0, 0)),
        out_shape=jax.ShapeDtypeStruct((B, 1024), jnp.float32),
    )(z, fc1_w)
    return pl.pallas_call(
        _head_body,
        in_specs=[
            pl.BlockSpec((B, 1024), lambda: (0, 0)),
            pl.BlockSpec((1024,), lambda: (0,)),
            pl.BlockSpec((1024, 16), lambda: (0, 0)),
            pl.BlockSpec((16,), lambda: (0,)),
        ],
        out_specs=pl.BlockSpec((B, 16), lambda: (0, 0)),
        out_shape=jax.ShapeDtypeStruct((B, 16), jnp.float32),
    )(acc, fc1_b, jnp.pad(fc2_w, ((0, 0), (0, 6)), constant_values=-1e30),
      jnp.pad(fc2_b, (0, 6)))[:, :10]


def kernel(x, edge_index, edge_attr, pos, batch, W1, root1, bias1, g1, be1, W2, root2, bias2, g2, be2, W3, root3, bias3, g3, be3, W4, root4, bias4, g4, be4, fc1_w, fc1_b, fc2_w, fc2_b):
    src = edge_index[0]
    dst = edge_index[1]
    emask = jnp.ones((src.shape[0],), x.dtype)
    h = jax.nn.elu(_spline_conv(x, src, dst, edge_attr, emask, W1, root1, bias1))
    h = _bn(h, g1, be1)
    c = _voxel(pos, batch, 20.0, 6)
    h, pos, batch, src, dst, emask, pseudo = _max_pool(c, B * 216, h, pos, src, dst, emask, 6)
    h = jax.nn.elu(_spline_conv(h, src, dst, pseudo, emask, W2, root2, bias2))
    h = _bn(h, g2, be2)
    c = _voxel(pos, batch, 30.0, 4)
    h, pos, batch, src, dst, emask, pseudo = _max_pool(c, B * 64, h, pos, src, dst, emask, 4)
    h = jax.nn.elu(_spline_conv(h, src, dst, pseudo, emask, W3, root3, bias3))
    h = _bn(h, g3, be3)
    c = _voxel(pos, batch, 50.0, 3)
    h, pos, batch, src, dst, emask, pseudo = _max_pool(c, B * 27, h, pos, src, dst, emask, 3)
    h = jax.nn.elu(_spline_conv(h, src, dst, pseudo, emask, W4, root4, bias4))
    h = _bn(h, g4, be4)
    cg = jnp.clip(jnp.floor(pos / 100.0).astype(jnp.int32), 0, 1)
    local = cg[:, 0] + cg[:, 1] * 2 + cg[:, 2] * 4
    slot = batch * 64 + local
    hf = jax.ops.segment_max(h, slot, num_segments=B * 64)
    hf = jnp.where(jnp.isfinite(hf), hf, 0.0)
    z = hf.reshape(B, 64 * 512)
    return _fc_head(z, fc1_w, fc1_b, fc2_w, fc2_b)


# speedup vs baseline: 1.2948x; 1.2947x over previous
import functools

import jax
import jax.numpy as jnp
from jax import lax
from jax.experimental import pallas as pl
from jax.experimental.pallas import tpu as pltpu
from jax.experimental.pallas import tpu_sc as plsc

K = 5
B = 8

_TAPK = 32
_CH = 32
_NTILES = 32
_NB = 1280
_WBR = {64: 128, 128: 64, 256: 32, 512: 16}


def _spline_basis(pseudo):
    v = jnp.clip(pseudo, 0.0, 1.0) * (K - 1)
    f = jnp.clip(jnp.floor(v), 0, K - 2)
    frac = v - f
    fi = f.astype(jnp.int32)
    bas, wis = [], []
    for b in range(8):
        w = jnp.ones((pseudo.shape[0],), pseudo.dtype)
        idx = jnp.zeros((pseudo.shape[0],), jnp.int32)
        for d in range(3):
            bit = (b >> d) & 1
            w = w * (frac[:, d] if bit else (1.0 - frac[:, d]))
            idx = idx + (fi[:, d] + bit) * (K ** d)
        bas.append(w)
        wis.append(idx)
    return jnp.stack(bas, 1), jnp.stack(wis, 1)


def _agg_body(C, N_pad, R_t, nch,
              ridx_hbm, oidx_hbm, scale_hbm, y_hbm, out_hbm,
              ridx_v, oidx_v, scale_v, stag_v, wb_v, acc_sh):
    wbr = _WBR[C]
    c = lax.axis_index("c")
    s = lax.axis_index("s")
    row0 = s * R_t

    def zrow(r, _):
        for cc in range(C // 16):
            wb_v[r, pl.ds(cc * 16, 16)] = jnp.zeros((16,), jnp.float32)
        return 0
    lax.fori_loop(0, wbr, zrow, 0)
    for j in range(R_t // wbr):
        pltpu.sync_copy(wb_v, acc_sh.at[pl.ds(row0 + j * wbr, wbr)])
    plsc.subcore_barrier()

    base = (c * 16 + s) * _NB

    def chunk(ch, _):
        off = base + ch * _CH
        pltpu.sync_copy(ridx_hbm.at[pl.ds(off, _CH)], ridx_v)
        pltpu.sync_copy(oidx_hbm.at[pl.ds(off, _CH)], oidx_v)
        pltpu.sync_copy(scale_hbm.at[pl.ds(off, _CH)], scale_v)

        def batch(j, _):
            pltpu.sync_copy(y_hbm.at[ridx_v.at[j]], stag_v)
            for half in range(_TAPK // 16):
                sv = scale_v[j, pl.ds(half * 16, 16)]
                for rr in range(16):
                    r = half * 16 + rr
                    sc = sv[rr]
                    for cc in range(C // 16):
                        sl = pl.ds(cc * 16, 16)
                        stag_v[r, sl] = stag_v[r, sl] * sc
            pltpu.sync_copy(stag_v, acc_sh.at[oidx_v.at[j]], add=True)
            return 0
        lax.fori_loop(0, _CH, batch, 0)
        return 0
    lax.fori_loop(0, nch, chunk, 0)
    plsc.subcore_barrier()

    for j in range(R_t // wbr):
        pltpu.sync_copy(acc_sh.at[pl.ds(row0 + j * wbr, wbr)], wb_v)
        pltpu.sync_copy(wb_v, out_hbm.at[pl.ds(c * N_pad + row0 + j * wbr, wbr)])


def _sc_agg(Y, ridx, oidx, scale, N_out):
    C = Y.shape[1]
    wbr = _WBR[C]
    N_pad = ((N_out + 16 * wbr - 1) // (16 * wbr)) * (16 * wbr)
    R_t = N_pad // 16
    cap = _NTILES * _NB * _TAPK
    T = ridx.shape[0]
    pad = cap - T
    ridx = jnp.pad(ridx, (0, pad)).reshape(_NTILES * _NB, _TAPK)
    oidx = jnp.pad(oidx, (0, pad)).reshape(_NTILES * _NB, _TAPK)
    scale = jnp.pad(scale, (0, pad)).reshape(_NTILES * _NB, _TAPK)
    nch = _NB // _CH

    body = functools.partial(_agg_body, C, N_pad, R_t, nch)
    out = pl.kernel(
        body,
        out_type=jax.ShapeDtypeStruct((2 * N_pad, C), jnp.float32),
        mesh=plsc.VectorSubcoreMesh(core_axis_name="c", subcore_axis_name="s"),
        compiler_params=pltpu.CompilerParams(use_tc_tiling_on_sc=False),
        scratch_types=[
            pltpu.VMEM((_CH, _TAPK), jnp.int32),
            pltpu.VMEM((_CH, _TAPK), jnp.int32),
            pltpu.VMEM((_CH, _TAPK), jnp.float32),
            pltpu.VMEM((_TAPK, C), jnp.float32),
            pltpu.VMEM((wbr, C), jnp.float32),
            pltpu.VMEM_SHARED((N_pad, C), jnp.float32),
        ],
    )(ridx, oidx, scale, Y)
    return out[:N_out] + out[N_pad:N_pad + N_out]


def _spline_conv_sc(x, src, dst, pseudo, emask, W, root, bias):
    N, Cin = x.shape
    Cout = W.shape[2]
    basis, wi = _spline_basis(pseudo)
    basis = basis * emask[:, None]
    if Cin == 1:
        Y = W.reshape(125, Cout)
        ridx = wi.T.reshape(-1)
        scale = (basis * x[src]).T.reshape(-1)
    else:
        Y = jnp.einsum('nc,kco->kno', x, W).reshape(125 * N, Cout)
        ridx = (wi * N + src[:, None]).T.reshape(-1)
        scale = basis.T.reshape(-1)
    oidx = jnp.broadcast_to(dst, (8, dst.shape[0])).reshape(-1)
    agg = _sc_agg(Y, ridx, oidx, scale, N)
    deg = jnp.zeros((N,), x.dtype).at[dst].add(emask)
    out = agg / jnp.clip(deg, 1.0, None)[:, None]
    return out + x @ root + bias


def _bn(x, g, b):
    m = jnp.mean(x, 0)
    v = jnp.var(x, 0)
    return g * (x - m) / jnp.sqrt(v + 1e-5) + b


def _voxel(pos, batch, size, nvox):
    c = jnp.clip(jnp.floor(pos / size).astype(jnp.int32), 0, nvox - 1)
    local = c[:, 0] + c[:, 1] * nvox + c[:, 2] * nvox * nvox
    return batch * (nvox ** 3) + local


def _max_pool(cluster, M, x, pos, src, dst, emask, nvox):
    xn = jax.ops.segment_max(x, cluster, num_segments=M)
    xn = jnp.where(jnp.isfinite(xn), xn, 0.0)
    cnt = jnp.zeros((M,), x.dtype).at[cluster].add(1.0)
    pn = jnp.zeros((M, 3), pos.dtype).at[cluster].add(pos) / jnp.clip(cnt, 1.0, None)[:, None]
    bnew = (jnp.arange(M, dtype=jnp.int32) // (nvox ** 3))
    sn = cluster[src]
    dn = cluster[dst]
    em = emask * (sn != dn).astype(x.dtype)
    delta = pn[dn] - pn[sn]
    mx = jnp.max(jnp.abs(delta))
    pseudo = delta / (2.0 * mx + 1e-12) + 0.5
    return xn, pn, bnew, sn, dn, em, pseudo


def _fc1_body(z_ref, w_ref, acc_ref):
    @pl.when(pl.program_id(0) == 0)
    def _():
        acc_ref[...] = jnp.zeros_like(acc_ref)
    acc_ref[...] += jnp.dot(z_ref[...], w_ref[...],
                            preferred_element_type=jnp.float32)


def _head_body(acc_ref, b1_ref, w2_ref, b2_ref, out_ref):
    h = acc_ref[...] + b1_ref[...]
    h = jnp.where(h > 0, h, jnp.exp(jnp.minimum(h, 0.0)) - 1.0)
    y = jnp.dot(h, w2_ref[...], preferred_element_type=jnp.float32) + b2_ref[...]
    mx = jnp.max(y, axis=1, keepdims=True)
    lse = jnp.log(jnp.sum(jnp.exp(y - mx), axis=1, keepdims=True)) + mx
    out_ref[...] = y - lse


def _fc_head(z, fc1_w, fc1_b, fc2_w, fc2_b):
    KC = 2048
    nk = z.shape[1] // KC
    acc = pl.pallas_call(
        _fc1_body,
        grid=(nk,),
        in_specs=[
            pl.BlockSpec((B, KC), lambda k: (0, k)),
            pl.BlockSpec((KC, 1024), lambda k: (k, 0)),
        ],
        out_specs=pl.BlockSpec((B, 1024), lambda k: (0, 0)),
        out_shape=jax.ShapeDtypeStruct((B, 1024), jnp.float32),
    )(z, fc1_w)
    return pl.pallas_call(
        _head_body,
        in_specs=[
            pl.BlockSpec((B, 1024), lambda: (0, 0)),
            pl.BlockSpec((1024,), lambda: (0,)),
            pl.BlockSpec((1024, 16), lambda: (0, 0)),
            pl.BlockSpec((16,), lambda: (0,)),
        ],
        out_specs=pl.BlockSpec((B, 16), lambda: (0, 0)),
        out_shape=jax.ShapeDtypeStruct((B, 16), jnp.float32),
    )(acc, fc1_b, jnp.pad(fc2_w, ((0, 0), (0, 6)), constant_values=-1e30),
      jnp.pad(fc2_b, (0, 6)))[:, :10]


def kernel(x, edge_index, edge_attr, pos, batch, W1, root1, bias1, g1, be1, W2, root2, bias2, g2, be2, W3, root3, bias3, g3, be3, W4, root4, bias4, g4, be4, fc1_w, fc1_b, fc2_w, fc2_b):
    src = edge_index[0]
    dst = edge_index[1]
    emask = jnp.ones((src.shape[0],), x.dtype)
    h = jax.nn.elu(_spline_conv_sc(x, src, dst, edge_attr, emask, W1, root1, bias1))
    h = _bn(h, g1, be1)
    c = _voxel(pos, batch, 20.0, 6)
    h, pos, batch, src, dst, emask, pseudo = _max_pool(c, B * 216, h, pos, src, dst, emask, 6)
    h = jax.nn.elu(_spline_conv_sc(h, src, dst, pseudo, emask, W2, root2, bias2))
    h = _bn(h, g2, be2)
    c = _voxel(pos, batch, 30.0, 4)
    h, pos, batch, src, dst, emask, pseudo = _max_pool(c, B * 64, h, pos, src, dst, emask, 4)
    h = jax.nn.elu(_spline_conv_sc(h, src, dst, pseudo, emask, W3, root3, bias3))
    h = _bn(h, g3, be3)
    c = _voxel(pos, batch, 50.0, 3)
    h, pos, batch, src, dst, emask, pseudo = _max_pool(c, B * 27, h, pos, src, dst, emask, 3)
    h = jax.nn.elu(_spline_conv_sc(h, src, dst, pseudo, emask, W4, root4, bias4))
    h = _bn(h, g4, be4)
    cg = jnp.clip(jnp.floor(pos / 100.0).astype(jnp.int32), 0, 1)
    local = cg[:, 0] + cg[:, 1] * 2 + cg[:, 2] * 4
    slot = batch * 64 + local
    hf = jax.ops.segment_max(h, slot, num_segments=B * 64)
    hf = jnp.where(jnp.isfinite(hf), hf, 0.0)
    z = hf.reshape(B, 64 * 512)
    return _fc_head(z, fc1_w, fc1_b, fc2_w, fc2_b)


# speedup vs baseline: 1.4592x; 1.1270x over previous
import functools

import jax
import jax.numpy as jnp
from jax import lax
from jax.experimental import pallas as pl
from jax.experimental.pallas import tpu as pltpu
from jax.experimental.pallas import tpu_sc as plsc

K = 5
B = 8

_TAPK = 32
_CH = 32
_NTILES = 32
_NB = 1280
_WBR = {64: 128, 128: 64, 256: 32, 512: 16}


def _spline_basis(pseudo):
    v = jnp.clip(pseudo, 0.0, 1.0) * (K - 1)
    f = jnp.clip(jnp.floor(v), 0, K - 2)
    frac = v - f
    fi = f.astype(jnp.int32)
    bas, wis = [], []
    for b in range(8):
        w = jnp.ones((pseudo.shape[0],), pseudo.dtype)
        idx = jnp.zeros((pseudo.shape[0],), jnp.int32)
        for d in range(3):
            bit = (b >> d) & 1
            w = w * (frac[:, d] if bit else (1.0 - frac[:, d]))
            idx = idx + (fi[:, d] + bit) * (K ** d)
        bas.append(w)
        wis.append(idx)
    return jnp.stack(bas, 1), jnp.stack(wis, 1)


def _scale_batch(scale_v, j, stag, C):
    for half in range(_TAPK // 16):
        sv = scale_v[j, pl.ds(half * 16, 16)]
        for rr in range(16):
            r = half * 16 + rr
            sc = sv[rr]
            for cc in range(C // 16):
                sl = pl.ds(cc * 16, 16)
                stag[r, sl] = stag[r, sl] * sc


def _agg_body(C, N_pad, R_t, nch,
              ridx_hbm, oidx_hbm, scale_hbm, y_hbm, out_hbm,
              ridx_v, oidx_v, scale_v, stag_a, stag_b, wb_v, acc_sh,
              sem_g, sem_s):
    wbr = _WBR[C]
    c = lax.axis_index("c")
    s = lax.axis_index("s")
    row0 = s * R_t

    def zrow(r, _):
        for cc in range(C // 16):
            wb_v[r, pl.ds(cc * 16, 16)] = jnp.zeros((16,), jnp.float32)
        return 0
    lax.fori_loop(0, wbr, zrow, 0)
    for j in range(R_t // wbr):
        pltpu.sync_copy(wb_v, acc_sh.at[pl.ds(row0 + j * wbr, wbr)])
    plsc.subcore_barrier()

    base = (c * 16 + s) * _NB

    def wait_gather(stag):
        pltpu.make_async_copy(y_hbm.at[ridx_v.at[0]], stag, sem_g).wait()

    def wait_scatter(stag):
        pltpu.make_async_copy(stag, acc_sh.at[oidx_v.at[0]], sem_s).wait()

    def chunk(ch, _):
        off = base + ch * _CH
        pltpu.sync_copy(ridx_hbm.at[pl.ds(off, _CH)], ridx_v)
        pltpu.sync_copy(oidx_hbm.at[pl.ds(off, _CH)], oidx_v)
        pltpu.sync_copy(scale_hbm.at[pl.ds(off, _CH)], scale_v)
        pltpu.async_copy(y_hbm.at[ridx_v.at[0]], stag_a, sem_g)

        def pair(g, _):
            j0 = 2 * g
            j1 = 2 * g + 1

            @pl.when(g > 0)
            def _():
                wait_scatter(stag_b)
            pltpu.async_copy(y_hbm.at[ridx_v.at[j1]], stag_b, sem_g)
            wait_gather(stag_a)
            _scale_batch(scale_v, j0, stag_a, C)
            pltpu.async_copy(stag_a, acc_sh.at[oidx_v.at[j0]], sem_s, add=True)
            wait_gather(stag_b)
            _scale_batch(scale_v, j1, stag_b, C)

            @pl.when(j1 + 1 < _CH)
            def _():
                wait_scatter(stag_a)
                pltpu.async_copy(y_hbm.at[ridx_v.at[j1 + 1]], stag_a, sem_g)
            pltpu.async_copy(stag_b, acc_sh.at[oidx_v.at[j1]], sem_s, add=True)
            return 0
        lax.fori_loop(0, _CH // 2, pair, 0)
        wait_scatter(stag_a)
        wait_scatter(stag_b)
        return 0
    lax.fori_loop(0, nch, chunk, 0)
    plsc.subcore_barrier()

    for j in range(R_t // wbr):
        pltpu.sync_copy(acc_sh.at[pl.ds(row0 + j * wbr, wbr)], wb_v)
        pltpu.sync_copy(wb_v, out_hbm.at[pl.ds(c * N_pad + row0 + j * wbr, wbr)])


def _sc_agg(Y, ridx, oidx, scale, N_out):
    C = Y.shape[1]
    wbr = _WBR[C]
    N_pad = ((N_out + 16 * wbr - 1) // (16 * wbr)) * (16 * wbr)
    R_t = N_pad // 16
    cap = _NTILES * _NB * _TAPK
    T = ridx.shape[0]
    pad = cap - T
    ridx = jnp.pad(ridx, (0, pad)).reshape(_NTILES * _NB, _TAPK)
    oidx = jnp.pad(oidx, (0, pad)).reshape(_NTILES * _NB, _TAPK)
    scale = jnp.pad(scale, (0, pad)).reshape(_NTILES * _NB, _TAPK)
    nch = _NB // _CH

    body = functools.partial(_agg_body, C, N_pad, R_t, nch)
    out = pl.kernel(
        body,
        out_type=jax.ShapeDtypeStruct((2 * N_pad, C), jnp.float32),
        mesh=plsc.VectorSubcoreMesh(core_axis_name="c", subcore_axis_name="s"),
        compiler_params=pltpu.CompilerParams(use_tc_tiling_on_sc=False),
        scratch_types=[
            pltpu.VMEM((_CH, _TAPK), jnp.int32),
            pltpu.VMEM((_CH, _TAPK), jnp.int32),
            pltpu.VMEM((_CH, _TAPK), jnp.float32),
            pltpu.VMEM((_TAPK, C), jnp.float32),
            pltpu.VMEM((_TAPK, C), jnp.float32),
            pltpu.VMEM((wbr, C), jnp.float32),
            pltpu.VMEM_SHARED((N_pad, C), jnp.float32),
            pltpu.SemaphoreType.DMA,
            pltpu.SemaphoreType.DMA,
        ],
    )(ridx, oidx, scale, Y)
    return out[:N_out] + out[N_pad:N_pad + N_out]


def _spline_conv_sc(x, src, dst, pseudo, emask, W, root, bias):
    N, Cin = x.shape
    Cout = W.shape[2]
    basis, wi = _spline_basis(pseudo)
    basis = basis * emask[:, None]
    if Cin == 1:
        Y = W.reshape(125, Cout)
        ridx = wi.T.reshape(-1)
        scale = (basis * x[src]).T.reshape(-1)
    else:
        Y = jnp.einsum('nc,kco->kno', x, W).reshape(125 * N, Cout)
        ridx = (wi * N + src[:, None]).T.reshape(-1)
        scale = basis.T.reshape(-1)
    oidx = jnp.broadcast_to(dst, (8, dst.shape[0])).reshape(-1)
    agg = _sc_agg(Y, ridx, oidx, scale, N)
    deg = jnp.zeros((N,), x.dtype).at[dst].add(emask)
    out = agg / jnp.clip(deg, 1.0, None)[:, None]
    return out + x @ root + bias


def _bn(x, g, b):
    m = jnp.mean(x, 0)
    v = jnp.var(x, 0)
    return g * (x - m) / jnp.sqrt(v + 1e-5) + b


def _voxel(pos, batch, size, nvox):
    c = jnp.clip(jnp.floor(pos / size).astype(jnp.int32), 0, nvox - 1)
    local = c[:, 0] + c[:, 1] * nvox + c[:, 2] * nvox * nvox
    return batch * (nvox ** 3) + local


def _max_pool(cluster, M, x, pos, src, dst, emask, nvox):
    xn = jax.ops.segment_max(x, cluster, num_segments=M)
    xn = jnp.where(jnp.isfinite(xn), xn, 0.0)
    cnt = jnp.zeros((M,), x.dtype).at[cluster].add(1.0)
    pn = jnp.zeros((M, 3), pos.dtype).at[cluster].add(pos) / jnp.clip(cnt, 1.0, None)[:, None]
    bnew = (jnp.arange(M, dtype=jnp.int32) // (nvox ** 3))
    sn = cluster[src]
    dn = cluster[dst]
    em = emask * (sn != dn).astype(x.dtype)
    delta = pn[dn] - pn[sn]
    mx = jnp.max(jnp.abs(delta))
    pseudo = delta / (2.0 * mx + 1e-12) + 0.5
    return xn, pn, bnew, sn, dn, em, pseudo


def _fc1_body(z_ref, w_ref, acc_ref):
    @pl.when(pl.program_id(0) == 0)
    def _():
        acc_ref[...] = jnp.zeros_like(acc_ref)
    acc_ref[...] += jnp.dot(z_ref[...], w_ref[...],
                            preferred_element_type=jnp.float32)


def _head_body(acc_ref, b1_ref, w2_ref, b2_ref, out_ref):
    h = acc_ref[...] + b1_ref[...]
    h = jnp.where(h > 0, h, jnp.exp(jnp.minimum(h, 0.0)) - 1.0)
    y = jnp.dot(h, w2_ref[...], preferred_element_type=jnp.float32) + b2_ref[...]
    mx = jnp.max(y, axis=1, keepdims=True)
    lse = jnp.log(jnp.sum(jnp.exp(y - mx), axis=1, keepdims=True)) + mx
    out_ref[...] = y - lse


def _fc_head(z, fc1_w, fc1_b, fc2_w, fc2_b):
    KC = 2048
    nk = z.shape[1] // KC
    acc = pl.pallas_call(
        _fc1_body,
        grid=(nk,),
        in_specs=[
            pl.BlockSpec((B, KC), lambda k: (0, k)),
            pl.BlockSpec((KC, 1024), lambda k: (k, 0)),
        ],
        out_specs=pl.BlockSpec((B, 1024), lambda k: (0, 0)),
        out_shape=jax.ShapeDtypeStruct((B, 1024), jnp.float32),
    )(z, fc1_w)
    return pl.pallas_call(
        _head_body,
        in_specs=[
            pl.BlockSpec((B, 1024), lambda: (0, 0)),
            pl.BlockSpec((1024,), lambda: (0,)),
            pl.BlockSpec((1024, 16), lambda: (0, 0)),
            pl.BlockSpec((16,), lambda: (0,)),
        ],
        out_specs=pl.BlockSpec((B, 16), lambda: (0, 0)),
        out_shape=jax.ShapeDtypeStruct((B, 16), jnp.float32),
    )(acc, fc1_b, jnp.pad(fc2_w, ((0, 0), (0, 6)), constant_values=-1e30),
      jnp.pad(fc2_b, (0, 6)))[:, :10]


def kernel(x, edge_index, edge_attr, pos, batch, W1, root1, bias1, g1, be1, W2, root2, bias2, g2, be2, W3, root3, bias3, g3, be3, W4, root4, bias4, g4, be4, fc1_w, fc1_b, fc2_w, fc2_b):
    src = edge_index[0]
    dst = edge_index[1]
    emask = jnp.ones((src.shape[0],), x.dtype)
    h = jax.nn.elu(_spline_conv_sc(x, src, dst, edge_attr, emask, W1, root1, bias1))
    h = _bn(h, g1, be1)
    c = _voxel(pos, batch, 20.0, 6)
    h, pos, batch, src, dst, emask, pseudo = _max_pool(c, B * 216, h, pos, src, dst, emask, 6)
    h = jax.nn.elu(_spline_conv_sc(h, src, dst, pseudo, emask, W2, root2, bias2))
    h = _bn(h, g2, be2)
    c = _voxel(pos, batch, 30.0, 4)
    h, pos, batch, src, dst, emask, pseudo = _max_pool(c, B * 64, h, pos, src, dst, emask, 4)
    h = jax.nn.elu(_spline_conv_sc(h, src, dst, pseudo, emask, W3, root3, bias3))
    h = _bn(h, g3, be3)
    c = _voxel(pos, batch, 50.0, 3)
    h, pos, batch, src, dst, emask, pseudo = _max_pool(c, B * 27, h, pos, src, dst, emask, 3)
    h = jax.nn.elu(_spline_conv_sc(h, src, dst, pseudo, emask, W4, root4, bias4))
    h = _bn(h, g4, be4)
    cg = jnp.clip(jnp.floor(pos / 100.0).astype(jnp.int32), 0, 1)
    local = cg[:, 0] + cg[:, 1] * 2 + cg[:, 2] * 4
    slot = batch * 64 + local
    hf = jax.ops.segment_max(h, slot, num_segments=B * 64)
    hf = jnp.where(jnp.isfinite(hf), hf, 0.0)
    z = hf.reshape(B, 64 * 512)
    return _fc_head(z, fc1_w, fc1_b, fc2_w, fc2_b)
